# Initial kernel scaffold; baseline (speedup 1.0000x reference)
import jax
import jax.numpy as jnp
from jax.experimental import pallas as pl


def kernel(x, labels, y, w1, b1, w2, b2):
    raise NotImplementedError("write your pallas kernel here")


# speedup vs baseline: 1.1816x; 1.1816x over previous
import functools

import jax
import jax.numpy as jnp
from jax import lax
from jax.experimental import pallas as pl
from jax.experimental.pallas import tpu as pltpu
from jax.experimental.pallas import tpu_sc as plsc

N, D, Y, S = 320000, 128, 16, 10000
NC, NS = 2, 16
HD = D // NC
RT = N // NS
C = 80
NCH = RT // C
SP = 10240
SEG_T = SP // NS

BR = 2000
SB = 2048


def _gate_body(x_ref, w1_ref, b1_ref, a_ref):
    xb = x_ref[...]
    t = jnp.dot(xb, w1_ref[...], preferred_element_type=jnp.float32)
    g = jax.nn.sigmoid(t + b1_ref[0, 0])
    a = g * xb
    a_ref[0] = a[:, :HD]
    a_ref[1] = a[:, HD:]


def _gate(x, w1, b1):
    return pl.pallas_call(
        _gate_body,
        grid=(N // BR,),
        in_specs=[
            pl.BlockSpec((BR, D), lambda i: (i, 0)),
            pl.BlockSpec((D, 1), lambda i: (0, 0)),
            pl.BlockSpec((1, 1), lambda i: (0, 0)),
        ],
        out_specs=pl.BlockSpec((NC, BR, HD), lambda i: (0, i, 0)),
        out_shape=jax.ShapeDtypeStruct((NC, N, HD), jnp.float32),
    )(x, w1, b1)


def _sc_scatter_body(a_hbm, y_hbm, lab_hbm, zres_hbm, zy_hbm,
                     res_hbm, ysc_hbm,
                     lab_c, a_v, y_v, res_sh, ysc_sh):
    c = lax.axis_index("c")
    s = lax.axis_index("s")
    r0 = s * SEG_T

    pltpu.sync_copy(zres_hbm.at[pl.ds(r0, SEG_T)], res_sh.at[pl.ds(r0, SEG_T)])
    pltpu.sync_copy(zy_hbm.at[pl.ds(r0, SEG_T)], ysc_sh.at[pl.ds(r0, SEG_T)])
    plsc.subcore_barrier()

    @pl.loop(0, NCH)
    def chunk(j):
        rbase = s * RT + j * C
        base = c * N + rbase
        pltpu.sync_copy(lab_hbm.at[pl.ds(rbase, C)], lab_c)
        pltpu.sync_copy(a_hbm.at[pl.ds(base, C)], a_v)
        pltpu.sync_copy(y_hbm.at[pl.ds(base, C)], y_v)
        pltpu.sync_copy(a_v, res_sh.at[lab_c], add=True)
        pltpu.sync_copy(y_v, ysc_sh.at[lab_c], add=True)

    plsc.subcore_barrier()

    pltpu.sync_copy(res_sh.at[pl.ds(r0, SEG_T)],
                    res_hbm.at[pl.ds(c * SP + r0, SEG_T)])
    pltpu.sync_copy(ysc_sh.at[pl.ds(r0, SEG_T)],
                    ysc_hbm.at[pl.ds(c * SP + r0, SEG_T)])


@functools.cache
def _get_sc_scatter():
    mesh = plsc.VectorSubcoreMesh(core_axis_name="c", subcore_axis_name="s")
    return functools.partial(
        pl.kernel,
        out_type=(
            jax.ShapeDtypeStruct((NC * SP, HD), jnp.float32),
            jax.ShapeDtypeStruct((NC * SP, Y), jnp.float32),
        ),
        mesh=mesh,
        compiler_params=pltpu.CompilerParams(use_tc_tiling_on_sc=False),
        scratch_types=[
            pltpu.VMEM((C,), jnp.int32),
            pltpu.VMEM((C, HD), jnp.float32),
            pltpu.VMEM((C, Y), jnp.float32),
            pltpu.VMEM_SHARED((SP, HD), jnp.float32),
            pltpu.VMEM_SHARED((SP, Y), jnp.float32),
        ],
    )(_sc_scatter_body)


def _head_body(res2_ref, ysc2_ref, w2_ref, b2_ref, out_ref, acc_ref):
    i = pl.program_id(0)

    @pl.when(i == 0)
    def _init():
        acc_ref[0] = 0.0
        acc_ref[1] = 0.0

    res = jnp.concatenate([res2_ref[0], res2_ref[1]], axis=1)
    ysum = ysc2_ref[0]
    cnt = ysc2_ref[1][:, 0:1]
    mask = (cnt > 0.0).astype(jnp.float32)
    cf = jnp.where(cnt > 0.0, cnt, 1.0)
    z = jnp.dot(res / cf, w2_ref[...], preferred_element_type=jnp.float32)
    pred = jax.nn.softmax(z + b2_ref[...], axis=1)
    logp = jax.nn.log_softmax(pred, axis=1)
    per = jnp.sum((ysum / cf) * logp, axis=1, keepdims=True)
    acc_ref[0] += jnp.sum(per * mask)
    acc_ref[1] += jnp.sum(mask)

    @pl.when(i == pl.num_programs(0) - 1)
    def _fin():
        out_ref[0, 0] = -acc_ref[0] / acc_ref[1]


def _head(res2, ysc2, w2, b2):
    return pl.pallas_call(
        _head_body,
        grid=(SP // SB,),
        in_specs=[
            pl.BlockSpec((NC, SB, HD), lambda i: (0, i, 0)),
            pl.BlockSpec((NC, SB, Y), lambda i: (0, i, 0)),
            pl.BlockSpec((D, Y), lambda i: (0, 0)),
            pl.BlockSpec((1, Y), lambda i: (0, 0)),
        ],
        out_specs=pl.BlockSpec(memory_space=pltpu.SMEM),
        out_shape=jax.ShapeDtypeStruct((1, 1), jnp.float32),
        scratch_shapes=[pltpu.SMEM((2,), jnp.float32)],
    )(res2, ysc2, w2, b2)


def kernel(x, labels, y, w1, b1, w2, b2):
    x = x.astype(jnp.float32)
    a = _gate(x, w1.astype(jnp.float32), b1.reshape(1, 1).astype(jnp.float32))
    lab = labels.astype(jnp.int32)
    yc = jnp.concatenate([y.astype(jnp.float32),
                          jnp.ones((N, Y), jnp.float32)], axis=0)
    zres = jnp.zeros((SP, HD), jnp.float32)
    zy = jnp.zeros((SP, Y), jnp.float32)
    res2, ysc2 = _get_sc_scatter()(
        a.reshape(NC * N, HD), yc, lab, zres, zy)
    res2 = res2.reshape(NC, SP, HD)
    ysc2 = ysc2.reshape(NC, SP, Y)
    out = _head(res2, ysc2, w2.astype(jnp.float32),
                b2.reshape(1, Y).astype(jnp.float32))
    return out[0, 0]


# speedup vs baseline: 1.9631x; 1.6614x over previous
import functools

import jax
import jax.numpy as jnp
from jax import lax
from jax.experimental import pallas as pl
from jax.experimental.pallas import tpu as pltpu
from jax.experimental.pallas import tpu_sc as plsc

N, D, Y, S = 320000, 128, 16, 10000
NC, NS = 2, 16
HN = N // NC
RTA = HN // NS
RTY = N // NS
C = 80
NCHA = RTA // C
NCHY = RTY // C
SP = 10240
SEG_T = SP // NS

BR = 2000
SB = 2048


def _gate_body(x_ref, w1_ref, b1_ref, a_ref):
    xb = x_ref[...]
    t = jnp.dot(xb, w1_ref[...], preferred_element_type=jnp.float32)
    a_ref[...] = jax.nn.sigmoid(t + b1_ref[0, 0]) * xb


def _gate(x, w1, b1):
    return pl.pallas_call(
        _gate_body,
        grid=(N // BR,),
        in_specs=[
            pl.BlockSpec((BR, D), lambda i: (i, 0)),
            pl.BlockSpec((D, 1), lambda i: (0, 0)),
            pl.BlockSpec((1, 1), lambda i: (0, 0)),
        ],
        out_specs=pl.BlockSpec((BR, D), lambda i: (i, 0)),
        out_shape=jax.ShapeDtypeStruct((N, D), jnp.float32),
    )(x, w1, b1)


def _sc_scatter_body(a_hbm, y_hbm, lab_hbm, zres_hbm, zy_hbm, ones_hbm,
                     res_hbm, ysc_hbm,
                     lab_c, a_v, y_v, res_sh, ysc_sh):
    c = lax.axis_index("c")
    s = lax.axis_index("s")
    r0 = s * SEG_T

    pltpu.sync_copy(zres_hbm.at[pl.ds(r0, SEG_T)], res_sh.at[pl.ds(r0, SEG_T)])
    pltpu.sync_copy(zy_hbm.at[pl.ds(r0, SEG_T)], ysc_sh.at[pl.ds(r0, SEG_T)])

    @pl.when(c == 1)
    def _ones():
        pltpu.sync_copy(ones_hbm, y_v)

    plsc.subcore_barrier()

    @pl.loop(0, NCHA)
    def chunk_a(j):
        rbase = c * HN + s * RTA + j * C
        pltpu.sync_copy(lab_hbm.at[pl.ds(rbase, C)], lab_c)
        pltpu.sync_copy(a_hbm.at[pl.ds(rbase, C)], a_v)
        pltpu.sync_copy(a_v, res_sh.at[lab_c], add=True)

    @pl.loop(0, NCHY)
    def chunk_y(j):
        rbase = s * RTY + j * C
        pltpu.sync_copy(lab_hbm.at[pl.ds(rbase, C)], lab_c)

        @pl.when(c == 0)
        def _ld_y():
            pltpu.sync_copy(y_hbm.at[pl.ds(rbase, C)], y_v)

        pltpu.sync_copy(y_v, ysc_sh.at[lab_c], add=True)

    plsc.subcore_barrier()

    pltpu.sync_copy(res_sh.at[pl.ds(r0, SEG_T)],
                    res_hbm.at[pl.ds(c * SP + r0, SEG_T)])
    pltpu.sync_copy(ysc_sh.at[pl.ds(r0, SEG_T)],
                    ysc_hbm.at[pl.ds(c * SP + r0, SEG_T)])


@functools.cache
def _get_sc_scatter():
    mesh = plsc.VectorSubcoreMesh(core_axis_name="c", subcore_axis_name="s")
    return functools.partial(
        pl.kernel,
        out_type=(
            jax.ShapeDtypeStruct((NC * SP, D), jnp.float32),
            jax.ShapeDtypeStruct((NC * SP, Y), jnp.float32),
        ),
        mesh=mesh,
        compiler_params=pltpu.CompilerParams(use_tc_tiling_on_sc=False),
        scratch_types=[
            pltpu.VMEM((C,), jnp.int32),
            pltpu.VMEM((C, D), jnp.float32),
            pltpu.VMEM((C, Y), jnp.float32),
            pltpu.VMEM_SHARED((SP, D), jnp.float32),
            pltpu.VMEM_SHARED((SP, Y), jnp.float32),
        ],
    )(_sc_scatter_body)


def _head_body(res2_ref, ysc2_ref, w2_ref, b2_ref, out_ref, acc_ref):
    i = pl.program_id(0)

    @pl.when(i == 0)
    def _init():
        acc_ref[0] = 0.0
        acc_ref[1] = 0.0

    res = res2_ref[0] + res2_ref[1]
    ysum = ysc2_ref[0]
    cnt = ysc2_ref[1][:, 0:1]
    mask = (cnt > 0.0).astype(jnp.float32)
    cf = jnp.where(cnt > 0.0, cnt, 1.0)
    z = jnp.dot(res / cf, w2_ref[...], preferred_element_type=jnp.float32)
    pred = jax.nn.softmax(z + b2_ref[...], axis=1)
    logp = jax.nn.log_softmax(pred, axis=1)
    per = jnp.sum((ysum / cf) * logp, axis=1, keepdims=True)
    acc_ref[0] += jnp.sum(per * mask)
    acc_ref[1] += jnp.sum(mask)

    @pl.when(i == pl.num_programs(0) - 1)
    def _fin():
        out_ref[0, 0] = -acc_ref[0] / acc_ref[1]


def _head(res2, ysc2, w2, b2):
    return pl.pallas_call(
        _head_body,
        grid=(SP // SB,),
        in_specs=[
            pl.BlockSpec((NC, SB, D), lambda i: (0, i, 0)),
            pl.BlockSpec((NC, SB, Y), lambda i: (0, i, 0)),
            pl.BlockSpec((D, Y), lambda i: (0, 0)),
            pl.BlockSpec((1, Y), lambda i: (0, 0)),
        ],
        out_specs=pl.BlockSpec(memory_space=pltpu.SMEM),
        out_shape=jax.ShapeDtypeStruct((1, 1), jnp.float32),
        scratch_shapes=[pltpu.SMEM((2,), jnp.float32)],
    )(res2, ysc2, w2, b2)


def kernel(x, labels, y, w1, b1, w2, b2):
    x = x.astype(jnp.float32)
    a = _gate(x, w1.astype(jnp.float32), b1.reshape(1, 1).astype(jnp.float32))
    lab = labels.astype(jnp.int32)
    yf = y.astype(jnp.float32)
    ones_c = jnp.ones((C, Y), jnp.float32)
    zres = jnp.zeros((SP, D), jnp.float32)
    zy = jnp.zeros((SP, Y), jnp.float32)
    res2, ysc2 = _get_sc_scatter()(a, yf, lab, zres, zy, ones_c)
    res2 = res2.reshape(NC, SP, D)
    ysc2 = ysc2.reshape(NC, SP, Y)
    out = _head(res2, ysc2, w2.astype(jnp.float32),
                b2.reshape(1, Y).astype(jnp.float32))
    return out[0, 0]


# speedup vs baseline: 2.0283x; 1.0333x over previous
import functools

import jax
import jax.numpy as jnp
from jax import lax
from jax.experimental import pallas as pl
from jax.experimental.pallas import tpu as pltpu
from jax.experimental.pallas import tpu_sc as plsc

N, D, Y, S = 320000, 128, 16, 10000
NC, NS = 2, 16
HN0 = 140800
HN1 = N - HN0
RTA0 = HN0 // NS
RTA1 = HN1 // NS
RTY = N // NS
C = 80
NCHA0 = RTA0 // C
NCHA1 = RTA1 // C
NCHY = RTY // C
SP = 10240
SEG_T = SP // NS

BR = 2000
SB = 2048


def _gate_body(x_ref, w1_ref, b1_ref, a_ref):
    xb = x_ref[...]
    t = jnp.dot(xb, w1_ref[...], preferred_element_type=jnp.float32)
    a_ref[...] = jax.nn.sigmoid(t + b1_ref[0, 0]) * xb


def _gate(x, w1, b1):
    return pl.pallas_call(
        _gate_body,
        grid=(N // BR,),
        in_specs=[
            pl.BlockSpec((BR, D), lambda i: (i, 0)),
            pl.BlockSpec((D, 1), lambda i: (0, 0)),
            pl.BlockSpec((1, 1), lambda i: (0, 0)),
        ],
        out_specs=pl.BlockSpec((BR, D), lambda i: (i, 0)),
        out_shape=jax.ShapeDtypeStruct((N, D), jnp.float32),
    )(x, w1, b1)


def _sc_scatter_body(a_hbm, y_hbm, lab_hbm, zres_hbm, zy_hbm, ones_hbm,
                     res_hbm, ysc_hbm,
                     lab_c, a_v, y_v, res_sh, ysc_sh):
    c = lax.axis_index("c")
    s = lax.axis_index("s")
    r0 = s * SEG_T

    pltpu.sync_copy(zres_hbm.at[pl.ds(r0, SEG_T)], res_sh.at[pl.ds(r0, SEG_T)])
    pltpu.sync_copy(zy_hbm.at[pl.ds(r0, SEG_T)], ysc_sh.at[pl.ds(r0, SEG_T)])

    @pl.when(c == 1)
    def _ones():
        pltpu.sync_copy(ones_hbm, y_v)

    plsc.subcore_barrier()

    start = c * HN0
    rta = jnp.where(c == 0, RTA0, RTA1)
    nch = jnp.where(c == 0, NCHA0, NCHA1)

    @pl.loop(0, NCHA1)
    def chunk_a(j):
        @pl.when(j < nch)
        def _do():
            rbase = start + s * rta + j * C
            pltpu.sync_copy(lab_hbm.at[pl.ds(rbase, C)], lab_c)
            pltpu.sync_copy(a_hbm.at[pl.ds(rbase, C)], a_v)
            pltpu.sync_copy(a_v, res_sh.at[lab_c], add=True)

    @pl.loop(0, NCHY)
    def chunk_y(j):
        rbase = s * RTY + j * C
        pltpu.sync_copy(lab_hbm.at[pl.ds(rbase, C)], lab_c)

        @pl.when(c == 0)
        def _ld_y():
            pltpu.sync_copy(y_hbm.at[pl.ds(rbase, C)], y_v)

        pltpu.sync_copy(y_v, ysc_sh.at[lab_c], add=True)

    plsc.subcore_barrier()

    pltpu.sync_copy(res_sh.at[pl.ds(r0, SEG_T)],
                    res_hbm.at[pl.ds(c * SP + r0, SEG_T)])
    pltpu.sync_copy(ysc_sh.at[pl.ds(r0, SEG_T)],
                    ysc_hbm.at[pl.ds(c * SP + r0, SEG_T)])


@functools.cache
def _get_sc_scatter():
    mesh = plsc.VectorSubcoreMesh(core_axis_name="c", subcore_axis_name="s")
    return functools.partial(
        pl.kernel,
        out_type=(
            jax.ShapeDtypeStruct((NC * SP, D), jnp.float32),
            jax.ShapeDtypeStruct((NC * SP, Y), jnp.float32),
        ),
        mesh=mesh,
        compiler_params=pltpu.CompilerParams(use_tc_tiling_on_sc=False),
        scratch_types=[
            pltpu.VMEM((C,), jnp.int32),
            pltpu.VMEM((C, D), jnp.float32),
            pltpu.VMEM((C, Y), jnp.float32),
            pltpu.VMEM_SHARED((SP, D), jnp.float32),
            pltpu.VMEM_SHARED((SP, Y), jnp.float32),
        ],
    )(_sc_scatter_body)


def _head_body(res2_ref, ysc2_ref, w2_ref, b2_ref, out_ref, acc_ref):
    i = pl.program_id(0)

    @pl.when(i == 0)
    def _init():
        acc_ref[0] = 0.0
        acc_ref[1] = 0.0

    res = res2_ref[0] + res2_ref[1]
    ysum = ysc2_ref[0]
    cnt = ysc2_ref[1][:, 0:1]
    mask = (cnt > 0.0).astype(jnp.float32)
    cf = jnp.where(cnt > 0.0, cnt, 1.0)
    z = jnp.dot(res / cf, w2_ref[...], preferred_element_type=jnp.float32)
    pred = jax.nn.softmax(z + b2_ref[...], axis=1)
    logp = jax.nn.log_softmax(pred, axis=1)
    per = jnp.sum((ysum / cf) * logp, axis=1, keepdims=True)
    acc_ref[0] += jnp.sum(per * mask)
    acc_ref[1] += jnp.sum(mask)

    @pl.when(i == pl.num_programs(0) - 1)
    def _fin():
        out_ref[0, 0] = -acc_ref[0] / acc_ref[1]


def _head(res2, ysc2, w2, b2):
    return pl.pallas_call(
        _head_body,
        grid=(SP // SB,),
        in_specs=[
            pl.BlockSpec((NC, SB, D), lambda i: (0, i, 0)),
            pl.BlockSpec((NC, SB, Y), lambda i: (0, i, 0)),
            pl.BlockSpec((D, Y), lambda i: (0, 0)),
            pl.BlockSpec((1, Y), lambda i: (0, 0)),
        ],
        out_specs=pl.BlockSpec(memory_space=pltpu.SMEM),
        out_shape=jax.ShapeDtypeStruct((1, 1), jnp.float32),
        scratch_shapes=[pltpu.SMEM((2,), jnp.float32)],
    )(res2, ysc2, w2, b2)


def kernel(x, labels, y, w1, b1, w2, b2):
    x = x.astype(jnp.float32)
    a = _gate(x, w1.astype(jnp.float32), b1.reshape(1, 1).astype(jnp.float32))
    lab = labels.astype(jnp.int32)
    yf = y.astype(jnp.float32)
    ones_c = jnp.ones((C, Y), jnp.float32)
    zres = jnp.zeros((SP, D), jnp.float32)
    zy = jnp.zeros((SP, Y), jnp.float32)
    res2, ysc2 = _get_sc_scatter()(a, yf, lab, zres, zy, ones_c)
    res2 = res2.reshape(NC, SP, D)
    ysc2 = ysc2.reshape(NC, SP, Y)
    out = _head(res2, ysc2, w2.astype(jnp.float32),
                b2.reshape(1, Y).astype(jnp.float32))
    return out[0, 0]
